# baseline (speedup 1.0000x reference)
import functools

import jax
import jax.numpy as jnp
from jax import lax
from jax.experimental import pallas as pl
from jax.experimental.pallas import tpu as pltpu
from jax.experimental.pallas import tpu_sc as plsc

N_TOK = 8192
DIM = 256
N_CODES = 8192

TM = 256
TN = 512
NI = N_TOK // TM
NJ = N_CODES // TN


def _scores(a, c):
    a2 = jnp.sum(a * a, axis=1, keepdims=True)
    b2 = jnp.sum(c * c, axis=1)[None, :]
    mm = lax.dot_general(a, c, (((1,), (1,)), ((), ())),
                         precision=lax.Precision.DEFAULT,
                         preferred_element_type=jnp.float32)
    d2 = (a2 + b2) - 2.0 * mm
    return jnp.maximum(d2, 0.0)


def _argmin_update(j, d2, idx_ref, bv_ref, bi_ref):
    bv = jnp.min(d2, axis=1)
    gidx = lax.broadcasted_iota(jnp.int32, d2.shape, 1) + j * TN
    cand = jnp.where(d2 == bv[:, None], gidx, jnp.int32(2**30))
    bi = jnp.min(cand, axis=1)

    @pl.when(j == 0)
    def _():
        bv_ref[...] = bv
        bi_ref[...] = bi

    @pl.when(j > 0)
    def _():
        better = bv < bv_ref[...]
        tie = (bv == bv_ref[...]) & (bi < bi_ref[...])
        upd = better | tie
        bi_ref[...] = jnp.where(upd, bi, bi_ref[...])
        bv_ref[...] = jnp.where(better, bv, bv_ref[...])

    @pl.when(j == NJ - 1)
    def _():
        idx_ref[...] = bi_ref[...]


def _argmin1_body(a_ref, c_ref, idx_ref, bv_ref, bi_ref):
    j = pl.program_id(1)
    d2 = _scores(a_ref[...], c_ref[...])
    _argmin_update(j, d2, idx_ref, bv_ref, bi_ref)


def _argmin2_body(x_ref, z_ref, c_ref, idx_ref, r_ref, bv_ref, bi_ref):
    j = pl.program_id(1)
    r = x_ref[...] - z_ref[...]

    @pl.when(j == 0)
    def _():
        r_ref[...] = r

    d2 = _scores(r, c_ref[...])
    _argmin_update(j, d2, idx_ref, bv_ref, bi_ref)


def _assemble_body(x_ref, z_ref, r_ref, iz_ref, out_ref):
    x = x_ref[...]
    z = z_ref[...]
    r = r_ref[...]
    iz = iz_ref[...]
    z_q = x + (z - x)
    inner_z_q = r + (iz - r)
    out_ref[...] = z_q + inner_z_q


_argmin1 = pl.pallas_call(
    _argmin1_body,
    grid=(NI, NJ),
    in_specs=[
        pl.BlockSpec((TM, DIM), lambda i, j: (i, 0)),
        pl.BlockSpec((TN, DIM), lambda i, j: (j, 0)),
    ],
    out_specs=pl.BlockSpec((TM,), lambda i, j: (i,)),
    out_shape=jax.ShapeDtypeStruct((N_TOK,), jnp.int32),
    scratch_shapes=[
        pltpu.VMEM((TM,), jnp.float32),
        pltpu.VMEM((TM,), jnp.int32),
    ],
    compiler_params=pltpu.CompilerParams(
        dimension_semantics=("arbitrary", "arbitrary")),
)

_argmin2 = pl.pallas_call(
    _argmin2_body,
    grid=(NI, NJ),
    in_specs=[
        pl.BlockSpec((TM, DIM), lambda i, j: (i, 0)),
        pl.BlockSpec((TM, DIM), lambda i, j: (i, 0)),
        pl.BlockSpec((TN, DIM), lambda i, j: (j, 0)),
    ],
    out_specs=[
        pl.BlockSpec((TM,), lambda i, j: (i,)),
        pl.BlockSpec((TM, DIM), lambda i, j: (i, 0)),
    ],
    out_shape=[
        jax.ShapeDtypeStruct((N_TOK,), jnp.int32),
        jax.ShapeDtypeStruct((N_TOK, DIM), jnp.float32),
    ],
    scratch_shapes=[
        pltpu.VMEM((TM,), jnp.float32),
        pltpu.VMEM((TM,), jnp.int32),
    ],
    compiler_params=pltpu.CompilerParams(
        dimension_semantics=("arbitrary", "arbitrary")),
)

_assemble = pl.pallas_call(
    _assemble_body,
    grid=(NI,),
    in_specs=[pl.BlockSpec((TM, DIM), lambda i: (i, 0))] * 4,
    out_specs=pl.BlockSpec((TM, DIM), lambda i: (i, 0)),
    out_shape=jax.ShapeDtypeStruct((N_TOK, DIM), jnp.float32),
)


@functools.cache
def _make_sc_gather():
    info = plsc.get_sparse_core_info()
    nw = info.num_cores * info.num_subcores
    rows_per_w = N_TOK // nw
    mesh = plsc.VectorSubcoreMesh(core_axis_name="c", subcore_axis_name="s")

    @functools.partial(
        pl.kernel,
        mesh=mesh,
        out_type=jax.ShapeDtypeStruct((N_TOK, DIM), jnp.float32),
        scratch_types=[
            pltpu.VMEM((rows_per_w,), jnp.int32),
            pltpu.VMEM((rows_per_w, DIM), jnp.float32),
            pltpu.SemaphoreType.DMA,
        ],
    )
    def gather_k(table_hbm, idx_hbm, out_hbm, idx_v, rows_v, sem):
        wid = lax.axis_index("s") * info.num_cores + lax.axis_index("c")
        base = wid * rows_per_w
        pltpu.sync_copy(idx_hbm.at[pl.ds(base, rows_per_w)], idx_v)
        pltpu.async_copy(table_hbm.at[idx_v], rows_v, sem).wait()
        pltpu.sync_copy(rows_v, out_hbm.at[pl.ds(base, rows_per_w)])

    return gather_k


def kernel(x, codebook, inner_codebook):
    sc_gather = _make_sc_gather()
    idx = _argmin1(x, codebook)
    z = sc_gather(codebook, idx)
    inner_idx, residual = _argmin2(x, z, inner_codebook)
    inner_z = sc_gather(inner_codebook, inner_idx)
    out0 = _assemble(x, z, residual, inner_z)
    return (out0, z, x, idx, inner_z, residual, inner_idx)


# speedup vs baseline: 1.0060x; 1.0060x over previous
import functools

import jax
import jax.numpy as jnp
from jax import lax
from jax.experimental import pallas as pl
from jax.experimental.pallas import tpu as pltpu
from jax.experimental.pallas import tpu_sc as plsc

N_TOK = 8192
DIM = 256
N_CODES = 8192

TM = 256
TN = 512
NI = N_TOK // TM
NJ = N_CODES // TN


def _scores(a, c):
    a2 = jnp.sum(a * a, axis=1, keepdims=True)
    b2 = jnp.sum(c * c, axis=1)[None, :]
    mm = lax.dot_general(a, c, (((1,), (1,)), ((), ())),
                         precision=lax.Precision.DEFAULT,
                         preferred_element_type=jnp.float32)
    d2 = (a2 + b2) - 2.0 * mm
    return jnp.maximum(d2, 0.0)


def _argmin_update(j, s, idx_ref, bv_ref, bi_ref):
    bv = jnp.min(s, axis=1)
    gidx = lax.broadcasted_iota(jnp.int32, s.shape, 1) + j * TN
    cand = jnp.where(s == bv[:, None], gidx, jnp.int32(2**30))
    bi = jnp.min(cand, axis=1)

    @pl.when(j == 0)
    def _():
        bv_ref[...] = bv
        bi_ref[...] = bi

    @pl.when(j > 0)
    def _():
        better = bv < bv_ref[...]
        tie = (bv == bv_ref[...]) & (bi < bi_ref[...])
        upd = better | tie
        bi_ref[...] = jnp.where(upd, bi, bi_ref[...])
        bv_ref[...] = jnp.where(better, bv, bv_ref[...])

    @pl.when(j == NJ - 1)
    def _():
        idx_ref[...] = bi_ref[...]


def _argmin1_body(a_ref, c_ref, idx_ref, bv_ref, bi_ref):
    j = pl.program_id(1)
    d2 = _scores(a_ref[...], c_ref[...])
    _argmin_update(j, d2, idx_ref, bv_ref, bi_ref)


def _argmin2_body(x_ref, z_ref, c_ref, idx_ref, r_ref, bv_ref, bi_ref):
    j = pl.program_id(1)
    r = x_ref[...] - z_ref[...]

    @pl.when(j == 0)
    def _():
        r_ref[...] = r

    d2 = _scores(r, c_ref[...])
    _argmin_update(j, d2, idx_ref, bv_ref, bi_ref)


def _assemble_body(x_ref, z_ref, r_ref, iz_ref, out_ref):
    x = x_ref[...]
    z = z_ref[...]
    r = r_ref[...]
    iz = iz_ref[...]
    z_q = x + (z - x)
    inner_z_q = r + (iz - r)
    out_ref[...] = z_q + inner_z_q


_argmin1 = pl.pallas_call(
    _argmin1_body,
    grid=(NI, NJ),
    in_specs=[
        pl.BlockSpec((TM, DIM), lambda i, j: (i, 0)),
        pl.BlockSpec((TN, DIM), lambda i, j: (j, 0)),
    ],
    out_specs=pl.BlockSpec((TM,), lambda i, j: (i,)),
    out_shape=jax.ShapeDtypeStruct((N_TOK,), jnp.int32),
    scratch_shapes=[
        pltpu.VMEM((TM,), jnp.float32),
        pltpu.VMEM((TM,), jnp.int32),
    ],
    compiler_params=pltpu.CompilerParams(
        dimension_semantics=("arbitrary", "arbitrary")),
)

_argmin2 = pl.pallas_call(
    _argmin2_body,
    grid=(NI, NJ),
    in_specs=[
        pl.BlockSpec((TM, DIM), lambda i, j: (i, 0)),
        pl.BlockSpec((TM, DIM), lambda i, j: (i, 0)),
        pl.BlockSpec((TN, DIM), lambda i, j: (j, 0)),
    ],
    out_specs=[
        pl.BlockSpec((TM,), lambda i, j: (i,)),
        pl.BlockSpec((TM, DIM), lambda i, j: (i, 0)),
    ],
    out_shape=[
        jax.ShapeDtypeStruct((N_TOK,), jnp.int32),
        jax.ShapeDtypeStruct((N_TOK, DIM), jnp.float32),
    ],
    scratch_shapes=[
        pltpu.VMEM((TM,), jnp.float32),
        pltpu.VMEM((TM,), jnp.int32),
    ],
    compiler_params=pltpu.CompilerParams(
        dimension_semantics=("arbitrary", "arbitrary")),
)

_assemble = pl.pallas_call(
    _assemble_body,
    grid=(NI,),
    in_specs=[pl.BlockSpec((TM, DIM), lambda i: (i, 0))] * 4,
    out_specs=pl.BlockSpec((TM, DIM), lambda i: (i, 0)),
    out_shape=jax.ShapeDtypeStruct((N_TOK, DIM), jnp.float32),
)


@functools.cache
def _make_sc_gather():
    info = plsc.get_sparse_core_info()
    nw = info.num_cores * info.num_subcores
    rows_per_w = N_TOK // nw
    mesh = plsc.VectorSubcoreMesh(core_axis_name="c", subcore_axis_name="s")

    @functools.partial(
        pl.kernel,
        mesh=mesh,
        out_type=jax.ShapeDtypeStruct((N_TOK, DIM), jnp.float32),
        scratch_types=[
            pltpu.VMEM((rows_per_w,), jnp.int32),
            pltpu.VMEM((rows_per_w, DIM), jnp.float32),
            pltpu.SemaphoreType.DMA,
        ],
    )
    def gather_k(table_hbm, idx_hbm, out_hbm, idx_v, rows_v, sem):
        wid = lax.axis_index("s") * info.num_cores + lax.axis_index("c")
        base = wid * rows_per_w
        pltpu.sync_copy(idx_hbm.at[pl.ds(base, rows_per_w)], idx_v)
        pltpu.async_copy(table_hbm.at[idx_v], rows_v, sem).wait()
        pltpu.sync_copy(rows_v, out_hbm.at[pl.ds(base, rows_per_w)])

    return gather_k


def kernel(x, codebook, inner_codebook):
    sc_gather = _make_sc_gather()
    idx = _argmin1(x, codebook)
    z = sc_gather(codebook, idx)
    inner_idx, residual = _argmin2(x, z, inner_codebook)
    inner_z = sc_gather(inner_codebook, inner_idx)
    out0 = _assemble(x, z, residual, inner_z)
    return (out0, z, x, idx, inner_z, residual, inner_idx)
